# baseline (speedup 1.0000x reference)
import functools

import jax
import jax.numpy as jnp
from jax import lax
from jax.experimental import pallas as pl
from jax.experimental.pallas import tpu as pltpu
from jax.experimental.pallas import tpu_sc as plsc

_K = 8


_BIG = 1 << 20
_NEG = -jnp.inf


def _topk_body(x_ref, idx_ref, *, n):
    rr = x_ref.shape[0]
    hs = 2 if rr >= 512 else 1
    for h in range(hs):
        _topk_half(x_ref, idx_ref, h, rr // hs, n=n)


def _topk_half(x_ref, idx_ref, h, r, *, n):
    xb = x_ref[pl.ds(h * r, r), :]
    nch = n // 128

    m = xb[:, 0:128]
    a = jnp.zeros((r, 128), jnp.int32)
    for s in range(1, nch):
        v = xb[:, 128 * s:128 * (s + 1)]
        gt = v > m
        a = jnp.where(gt, s, a)
        m = jnp.where(gt, v, m)
    lane_iota = jax.lax.broadcasted_iota(jnp.int32, (r, 128), 1)
    e = a * 128 + lane_iota

    lane_cols = []
    for _ in range(_K):
        mx = jnp.max(m, axis=1, keepdims=True)
        cand_e = jnp.where(m == mx, e, _BIG)
        sel_e = jnp.min(cand_e, axis=1, keepdims=True)
        sel_lane = jnp.bitwise_and(sel_e, 127)
        lane_cols.append(sel_lane)
        kill = lane_iota == sel_lane
        m = jnp.where(kill, _NEG, m)
        e = jnp.where(kill, _BIG, e)
    sel_lanes = jnp.concatenate(lane_cols, axis=1)

    cands = [
        jnp.take_along_axis(xb[:, 128 * s:128 * (s + 1)], sel_lanes, axis=1)
        for s in range(nch)
    ]
    cand = jnp.concatenate(cands, axis=1)
    cand_idx = jnp.concatenate(
        [sel_lanes + 128 * s for s in range(nch)], axis=1)

    out_cols = []
    for _ in range(_K):
        mx = jnp.max(cand, axis=1, keepdims=True)
        ce = jnp.where(cand == mx, cand_idx, _BIG)
        sel = jnp.min(ce, axis=1, keepdims=True)
        out_cols.append(sel)
        cand = jnp.where(cand_idx == sel, _NEG, cand)
    idx_ref[pl.ds(h * r, r), :] = jnp.concatenate(out_cols, axis=1)


def _sort_body(i_ref, rid_ref, *, b, c, n):
    a = i_ref[...]
    m = a.shape[1]
    iota0 = jax.lax.broadcasted_iota(jnp.int32, a.shape, 0)
    k = 2
    while k <= b:
        j = k // 2
        while j >= 1:
            g = b // (2 * j)
            a4 = a.reshape(g, 2, j, m)
            ap = jnp.concatenate([a4[:, 1:2], a4[:, 0:1]], axis=1).reshape(b, m)
            up = (iota0 & k) == 0
            low = (iota0 & j) == 0
            take_min = up == low
            a = jnp.where(take_min, jnp.minimum(a, ap), jnp.maximum(a, ap))
            j //= 2
        k *= 2
    col = jax.lax.broadcasted_iota(jnp.int32, a.shape, 1)
    row = iota0 * c + jnp.right_shift(col, 3)
    rid_ref[...] = a + row * n


def _sc_gather(table, rid, *, total):
    nw = 32
    per = total // nw

    mesh = plsc.VectorSubcoreMesh(core_axis_name="c", subcore_axis_name="s")

    @functools.partial(
        pl.kernel, mesh=mesh,
        out_type=jax.ShapeDtypeStruct((total,), jnp.float32),
        scratch_types=[
            pltpu.VMEM((per,), jnp.int32),
            pltpu.VMEM((per,), jnp.float32),
            pltpu.SemaphoreType.DMA,
        ],
    )
    def k(table_hbm, rid_hbm, out_hbm, rid_v, out_v, sem):
        wid = lax.axis_index("s") * 2 + lax.axis_index("c")
        base = wid * per
        pltpu.sync_copy(rid_hbm.at[pl.ds(base, per)], rid_v)
        pltpu.async_copy(table_hbm.at[rid_v], out_v, sem).wait()
        pltpu.sync_copy(out_v, out_hbm.at[pl.ds(base, per)])

    return k(table, rid)


def _gather_body(x_ref, idx_ref, o_ref):
    xb = x_ref[...]
    ib = idx_ref[...]
    iota = jax.lax.broadcasted_iota(jnp.int32, xb.shape, 1)
    cols = []
    for j in range(_K):
        sel = ib[:, j:j + 1]
        v = jnp.sum(jnp.where(iota == sel, xb, 0.0), axis=1, keepdims=True)
        cols.append(v)
    o_ref[...] = jnp.concatenate(cols, axis=1)


def kernel(x):
    b, c, n = x.shape
    bc = b * c
    x2 = x.reshape(bc, n)
    r = min(1024, bc)

    idx = pl.pallas_call(
        functools.partial(_topk_body, n=n),
        grid=(bc // r,),
        in_specs=[pl.BlockSpec((r, n), lambda i: (i, 0))],
        out_specs=pl.BlockSpec((r, _K), lambda i: (i, 0)),
        out_shape=jax.ShapeDtypeStruct((bc, _K), jnp.int32),
        compiler_params=pltpu.CompilerParams(
            dimension_semantics=("parallel",)),
    )(x2)

    idx_b = idx.reshape(b, c * _K)
    spec = pl.BlockSpec((b, c * _K), lambda: (0, 0))
    rid = pl.pallas_call(
        functools.partial(_sort_body, b=b, c=c, n=n),
        in_specs=[spec],
        out_specs=spec,
        out_shape=jax.ShapeDtypeStruct((b, c * _K), jnp.int32),
    )(idx_b)

    total = bc * _K
    out = _sc_gather(x.reshape(bc * n), rid.reshape(total), total=total)
    return out.reshape(b, c, _K)


# speedup vs baseline: 1.0893x; 1.0893x over previous
import functools

import jax
import jax.numpy as jnp
from jax import lax
from jax.experimental import pallas as pl
from jax.experimental.pallas import tpu as pltpu
from jax.experimental.pallas import tpu_sc as plsc

_K = 8


_BIG = 1 << 20
_NEG = -jnp.inf


def _topk_body(x_ref, idx_ref, *, n):
    rr = x_ref.shape[0]
    hs = 1
    for h in range(hs):
        _topk_half(x_ref, idx_ref, h, rr // hs, n=n)


def _topk_half(x_ref, idx_ref, h, r, *, n):
    xb = x_ref[pl.ds(h * r, r), :]
    nch = n // 128

    m = xb[:, 0:128]
    a = jnp.zeros((r, 128), jnp.int32)
    for s in range(1, nch):
        v = xb[:, 128 * s:128 * (s + 1)]
        gt = v > m
        a = jnp.where(gt, s, a)
        m = jnp.where(gt, v, m)
    lane_iota = jax.lax.broadcasted_iota(jnp.int32, (r, 128), 1)
    e = a * 128 + lane_iota

    lane_cols = []
    for _ in range(_K):
        mx = jnp.max(m, axis=1, keepdims=True)
        cand_e = jnp.where(m == mx, e, _BIG)
        sel_e = jnp.min(cand_e, axis=1, keepdims=True)
        sel_lane = jnp.bitwise_and(sel_e, 127)
        lane_cols.append(sel_lane)
        kill = lane_iota == sel_lane
        m = jnp.where(kill, _NEG, m)
        e = jnp.where(kill, _BIG, e)
    sel_lanes = jnp.concatenate(lane_cols, axis=1)

    cands = [
        jnp.take_along_axis(xb[:, 128 * s:128 * (s + 1)], sel_lanes, axis=1)
        for s in range(nch)
    ]
    cand = jnp.concatenate(cands, axis=1)
    cand_idx = jnp.concatenate(
        [sel_lanes + 128 * s for s in range(nch)], axis=1)

    out_cols = []
    for _ in range(_K):
        mx = jnp.max(cand, axis=1, keepdims=True)
        ce = jnp.where(cand == mx, cand_idx, _BIG)
        sel = jnp.min(ce, axis=1, keepdims=True)
        out_cols.append(sel)
        cand = jnp.where(cand_idx == sel, _NEG, cand)
    idx_ref[pl.ds(h * r, r), :] = jnp.concatenate(out_cols, axis=1)


def _sort_body(i_ref, rid_ref, *, b, c, n):
    a = i_ref[...]
    m = a.shape[1]
    iota0 = jax.lax.broadcasted_iota(jnp.int32, a.shape, 0)
    k = 2
    while k <= b:
        j = k // 2
        while j >= 1:
            g = b // (2 * j)
            a4 = a.reshape(g, 2, j, m)
            ap = jnp.concatenate([a4[:, 1:2], a4[:, 0:1]], axis=1).reshape(b, m)
            up = (iota0 & k) == 0
            low = (iota0 & j) == 0
            take_min = up == low
            a = jnp.where(take_min, jnp.minimum(a, ap), jnp.maximum(a, ap))
            j //= 2
        k *= 2
    col = jax.lax.broadcasted_iota(jnp.int32, a.shape, 1)
    row = iota0 * c + jnp.right_shift(col, 3)
    rid_ref[...] = a + row * n


def _sc_gather(table, rid, *, total):
    nw = 32
    per = total // nw

    mesh = plsc.VectorSubcoreMesh(core_axis_name="c", subcore_axis_name="s")

    @functools.partial(
        pl.kernel, mesh=mesh,
        out_type=jax.ShapeDtypeStruct((total,), jnp.float32),
        scratch_types=[
            pltpu.VMEM((per,), jnp.int32),
            pltpu.VMEM((per,), jnp.float32),
            pltpu.SemaphoreType.DMA,
        ],
    )
    def k(table_hbm, rid_hbm, out_hbm, rid_v, out_v, sem):
        wid = lax.axis_index("s") * 2 + lax.axis_index("c")
        base = wid * per
        pltpu.sync_copy(rid_hbm.at[pl.ds(base, per)], rid_v)
        pltpu.async_copy(table_hbm.at[rid_v], out_v, sem).wait()
        pltpu.sync_copy(out_v, out_hbm.at[pl.ds(base, per)])

    return k(table, rid)


def _gather_body(x_ref, idx_ref, o_ref):
    xb = x_ref[...]
    ib = idx_ref[...]
    iota = jax.lax.broadcasted_iota(jnp.int32, xb.shape, 1)
    cols = []
    for j in range(_K):
        sel = ib[:, j:j + 1]
        v = jnp.sum(jnp.where(iota == sel, xb, 0.0), axis=1, keepdims=True)
        cols.append(v)
    o_ref[...] = jnp.concatenate(cols, axis=1)


def kernel(x):
    b, c, n = x.shape
    bc = b * c
    x2 = x.reshape(bc, n)
    r = min(1024, bc)

    idx = pl.pallas_call(
        functools.partial(_topk_body, n=n),
        grid=(bc // r,),
        in_specs=[pl.BlockSpec((r, n), lambda i: (i, 0))],
        out_specs=pl.BlockSpec((r, _K), lambda i: (i, 0)),
        out_shape=jax.ShapeDtypeStruct((bc, _K), jnp.int32),
        compiler_params=pltpu.CompilerParams(
            dimension_semantics=("parallel",)),
    )(x2)

    idx_b = idx.reshape(b, c * _K)
    spec = pl.BlockSpec((b, c * _K), lambda: (0, 0))
    rid = pl.pallas_call(
        functools.partial(_sort_body, b=b, c=c, n=n),
        in_specs=[spec],
        out_specs=spec,
        out_shape=jax.ShapeDtypeStruct((b, c * _K), jnp.int32),
    )(idx_b)

    total = bc * _K
    out = _sc_gather(x.reshape(bc * n), rid.reshape(total), total=total)
    return out.reshape(b, c, _K)


# speedup vs baseline: 1.0896x; 1.0003x over previous
import functools

import jax
import jax.numpy as jnp
from jax import lax
from jax.experimental import pallas as pl
from jax.experimental.pallas import tpu as pltpu
from jax.experimental.pallas import tpu_sc as plsc

_K = 8


_BIG = 1 << 20
_NEG = -jnp.inf


def _topk_body(x_ref, idx_ref, *, n):
    rr = x_ref.shape[0]
    hs = 1
    for h in range(hs):
        _topk_half(x_ref, idx_ref, h, rr // hs, n=n)


def _topk_half(x_ref, idx_ref, h, r, *, n):
    nch = n // 128
    rows = pl.ds(h * r, r)

    m = x_ref[rows, 0:128]
    a = jnp.zeros((r, 128), jnp.int32)
    for s in range(1, nch):
        v = x_ref[rows, 128 * s:128 * (s + 1)]
        gt = v > m
        a = jnp.where(gt, s, a)
        m = jnp.where(gt, v, m)
    lane_iota = jax.lax.broadcasted_iota(jnp.int32, (r, 128), 1)
    e = a * 128 + lane_iota

    lane_cols = []
    for _ in range(_K):
        mx = jnp.max(m, axis=1, keepdims=True)
        cand_e = jnp.where(m == mx, e, _BIG)
        sel_e = jnp.min(cand_e, axis=1, keepdims=True)
        sel_lane = jnp.bitwise_and(sel_e, 127)
        lane_cols.append(sel_lane)
        kill = lane_iota == sel_lane
        m = jnp.where(kill, _NEG, m)
        e = jnp.where(kill, _BIG, e)
    sel_lanes = jnp.concatenate(lane_cols, axis=1)

    cands = [
        jnp.take_along_axis(x_ref[rows, 128 * s:128 * (s + 1)], sel_lanes,
                            axis=1)
        for s in range(nch)
    ]
    cand = jnp.concatenate(cands, axis=1)
    cand_idx = jnp.concatenate(
        [sel_lanes + 128 * s for s in range(nch)], axis=1)

    out_cols = []
    for _ in range(_K):
        mx = jnp.max(cand, axis=1, keepdims=True)
        ce = jnp.where(cand == mx, cand_idx, _BIG)
        sel = jnp.min(ce, axis=1, keepdims=True)
        out_cols.append(sel)
        cand = jnp.where(cand_idx == sel, _NEG, cand)
    idx_ref[pl.ds(h * r, r), :] = jnp.concatenate(out_cols, axis=1)


def _sort_body(i_ref, rid_ref, *, b, c, n):
    a = i_ref[...]
    m = a.shape[1]
    iota0 = jax.lax.broadcasted_iota(jnp.int32, a.shape, 0)
    k = 2
    while k <= b:
        j = k // 2
        while j >= 1:
            g = b // (2 * j)
            a4 = a.reshape(g, 2, j, m)
            ap = jnp.concatenate([a4[:, 1:2], a4[:, 0:1]], axis=1).reshape(b, m)
            up = (iota0 & k) == 0
            low = (iota0 & j) == 0
            take_min = up == low
            a = jnp.where(take_min, jnp.minimum(a, ap), jnp.maximum(a, ap))
            j //= 2
        k *= 2
    col = jax.lax.broadcasted_iota(jnp.int32, a.shape, 1)
    row = iota0 * c + jnp.right_shift(col, 3)
    rid_ref[...] = a + row * n


def _sc_gather(table, rid, *, total):
    nw = 32
    per = total // nw

    mesh = plsc.VectorSubcoreMesh(core_axis_name="c", subcore_axis_name="s")

    @functools.partial(
        pl.kernel, mesh=mesh,
        out_type=jax.ShapeDtypeStruct((total,), jnp.float32),
        scratch_types=[
            pltpu.VMEM((per,), jnp.int32),
            pltpu.VMEM((per,), jnp.float32),
            pltpu.SemaphoreType.DMA,
        ],
    )
    def k(table_hbm, rid_hbm, out_hbm, rid_v, out_v, sem):
        wid = lax.axis_index("s") * 2 + lax.axis_index("c")
        base = wid * per
        pltpu.sync_copy(rid_hbm.at[pl.ds(base, per)], rid_v)
        pltpu.async_copy(table_hbm.at[rid_v], out_v, sem).wait()
        pltpu.sync_copy(out_v, out_hbm.at[pl.ds(base, per)])

    return k(table, rid)


def _gather_body(x_ref, idx_ref, o_ref):
    xb = x_ref[...]
    ib = idx_ref[...]
    iota = jax.lax.broadcasted_iota(jnp.int32, xb.shape, 1)
    cols = []
    for j in range(_K):
        sel = ib[:, j:j + 1]
        v = jnp.sum(jnp.where(iota == sel, xb, 0.0), axis=1, keepdims=True)
        cols.append(v)
    o_ref[...] = jnp.concatenate(cols, axis=1)


def kernel(x):
    b, c, n = x.shape
    bc = b * c
    x2 = x.reshape(bc, n)
    r = min(1024, bc)

    idx = pl.pallas_call(
        functools.partial(_topk_body, n=n),
        grid=(bc // r,),
        in_specs=[pl.BlockSpec((r, n), lambda i: (i, 0))],
        out_specs=pl.BlockSpec((r, _K), lambda i: (i, 0)),
        out_shape=jax.ShapeDtypeStruct((bc, _K), jnp.int32),
        compiler_params=pltpu.CompilerParams(
            dimension_semantics=("parallel",)),
    )(x2)

    idx_b = idx.reshape(b, c * _K)
    spec = pl.BlockSpec((b, c * _K), lambda: (0, 0))
    rid = pl.pallas_call(
        functools.partial(_sort_body, b=b, c=c, n=n),
        in_specs=[spec],
        out_specs=spec,
        out_shape=jax.ShapeDtypeStruct((b, c * _K), jnp.int32),
    )(idx_b)

    total = bc * _K
    out = _sc_gather(x.reshape(bc * n), rid.reshape(total), total=total)
    return out.reshape(b, c, _K)
